# Initial kernel scaffold; baseline (speedup 1.0000x reference)
import jax
import jax.numpy as jnp
from jax.experimental import pallas as pl


def kernel(xyz, occs, params):
    raise NotImplementedError("write your pallas kernel here")


# speedup vs baseline: 1.0516x; 1.0516x over previous
import functools

import jax
import jax.numpy as jnp
from jax.experimental import pallas as pl

_NOCC = 2


def _conv_kern(w_ref, b_ref, x_ref, y_ref, st_ref):
    bi = pl.program_id(0)
    mi = pl.program_id(1)

    @pl.when((bi == 0) & (mi == 0))
    def _():
        st_ref[...] = jnp.zeros_like(st_ref)

    x = x_ref[0]
    w = w_ref[...]
    y = jnp.dot(w, x, preferred_element_type=jnp.float32) + b_ref[...]
    y_ref[0] = y
    s = jnp.sum(y, axis=1)
    sq = jnp.sum(y * y, axis=1)
    st_ref[...] += jnp.stack([s, sq])


def _pick_tm(m):
    for t in (8192, 4096, 2048, 1024, 512, 256, 128):
        if m % t == 0:
            return t
    return m


def _conv_stats(x, w, bias):
    B_, C, M = x.shape
    O = w.shape[0]
    tm = _pick_tm(M)
    grid = (B_, M // tm)
    y, st = pl.pallas_call(
        _conv_kern,
        grid=grid,
        in_specs=[
            pl.BlockSpec((O, C), lambda b, m: (0, 0)),
            pl.BlockSpec((O, 1), lambda b, m: (0, 0)),
            pl.BlockSpec((1, C, tm), lambda b, m: (b, 0, m)),
        ],
        out_specs=[
            pl.BlockSpec((1, O, tm), lambda b, m: (b, 0, m)),
            pl.BlockSpec((2, O), lambda b, m: (0, 0)),
        ],
        out_shape=[
            jax.ShapeDtypeStruct((B_, O, M), jnp.float32),
            jax.ShapeDtypeStruct((2, O), jnp.float32),
        ],
    )(w, bias.reshape(O, 1), x)
    return y, st


def _bn_coeffs(st, count, g, be):
    mean = st[0] / count
    var = st[1] / count - mean * mean
    rstd = g / jnp.sqrt(var + 1e-5)
    a = rstd
    c = be - mean * rstd
    return a.reshape(-1, 1), c.reshape(-1, 1)


def _nr_kern(a_ref, c_ref, y_ref, o_ref):
    o_ref[0] = jnp.maximum(y_ref[0] * a_ref[...] + c_ref[...], 0.0)


def _norm_relu(y, a, c):
    B_, O, M = y.shape
    tm = _pick_tm(M)
    return pl.pallas_call(
        _nr_kern,
        grid=(B_, M // tm),
        in_specs=[
            pl.BlockSpec((O, 1), lambda b, m: (0, 0)),
            pl.BlockSpec((O, 1), lambda b, m: (0, 0)),
            pl.BlockSpec((1, O, tm), lambda b, m: (b, 0, m)),
        ],
        out_specs=pl.BlockSpec((1, O, tm), lambda b, m: (b, 0, m)),
        out_shape=jax.ShapeDtypeStruct((B_, O, M), jnp.float32),
    )(a, c, y)


def _nrmaxk_kern(a_ref, c_ref, y_ref, o_ref):
    m = jnp.max(y_ref[0], axis=1)
    o_ref[0] = jnp.maximum(m * a_ref[...] + c_ref[...], 0.0)


def _norm_relu_maxk(y, a, c, K, S):
    B_, O, _ = y.shape
    y4 = y.reshape(B_, O, K, S)
    ts = _pick_tm(S)
    while ts > 128 and O * K * ts * 4 > (8 << 20):
        ts //= 2
    return pl.pallas_call(
        _nrmaxk_kern,
        grid=(B_, S // ts),
        in_specs=[
            pl.BlockSpec((O, 1), lambda b, s: (0, 0)),
            pl.BlockSpec((O, 1), lambda b, s: (0, 0)),
            pl.BlockSpec((1, O, K, ts), lambda b, s: (b, 0, 0, s)),
        ],
        out_specs=pl.BlockSpec((1, O, ts), lambda b, s: (b, 0, s)),
        out_shape=jax.ShapeDtypeStruct((B_, O, S), jnp.float32),
    )(a, c, y4)


def _nrmaxlane_kern(a_ref, c_ref, y_ref, o_ref):
    m = jnp.max(y_ref[0], axis=1, keepdims=True)
    o_ref[0] = jnp.maximum(m * a_ref[...] + c_ref[...], 0.0)


def _norm_relu_maxlane(y, a, c):
    B_, O, M = y.shape
    return pl.pallas_call(
        _nrmaxlane_kern,
        grid=(B_,),
        in_specs=[
            pl.BlockSpec((O, 1), lambda b: (0, 0)),
            pl.BlockSpec((O, 1), lambda b: (0, 0)),
            pl.BlockSpec((1, O, M), lambda b: (b, 0, 0)),
        ],
        out_specs=pl.BlockSpec((1, O, 1), lambda b: (b, 0, 0)),
        out_shape=jax.ShapeDtypeStruct((B_, O, 1), jnp.float32),
    )(a, c, y)


def _mlp_flat(x, layers):
    B_, _, M = x.shape
    h = x
    for li, p in enumerate(layers):
        y, st = _conv_stats(h, p['w'], p['b'])
        a, c = _bn_coeffs(st, B_ * M, p['g'], p['be'])
        if li < len(layers) - 1:
            h = _norm_relu(y, a, c)
        else:
            return y, a, c
    raise AssertionError


def _mlp2d_max(x4, layers):
    B_, C, K, S = x4.shape
    y, a, c = _mlp_flat(x4.reshape(B_, C, K * S), layers)
    return _norm_relu_maxk(y, a, c, K, S)


def _mlp1d(x, layers):
    y, a, c = _mlp_flat(x, layers)
    return _norm_relu(y, a, c)


def _fps_kern(x_ref, o_ref, *, npoint):
    xyz = x_ref[...]
    B_, _, N_ = xyz.shape
    iota = jax.lax.broadcasted_iota(jnp.int32, (B_, 1, N_), 2)
    piota = jax.lax.broadcasted_iota(jnp.int32, (1, 1, npoint), 2)

    def body(i, st):
        dist_min, far, acc = st
        onehot = (iota == far[:, :, None]).astype(jnp.float32)
        cen = jnp.sum(xyz * onehot, axis=2, keepdims=True)
        acc = jnp.where(piota == i, cen, acc)
        d = jnp.sum((xyz - cen) ** 2, axis=1, keepdims=True)
        dist_min = jnp.minimum(dist_min, d)
        far = jnp.argmax(dist_min[:, 0, :], axis=1).astype(jnp.int32)[:, None]
        return dist_min, far, acc

    dist0 = jnp.full((B_, 1, N_), 1e10, jnp.float32)
    far0 = jnp.zeros((B_, 1), jnp.int32)
    acc0 = jnp.zeros((B_, 3, npoint), jnp.float32)
    _, _, acc = jax.lax.fori_loop(0, npoint, body, (dist0, far0, acc0))
    o_ref[...] = acc


def _fps_newxyz(xyz_bcn, npoint):
    B_, _, N_ = xyz_bcn.shape
    return pl.pallas_call(
        functools.partial(_fps_kern, npoint=npoint),
        out_shape=jax.ShapeDtypeStruct((B_, 3, npoint), jnp.float32),
    )(xyz_bcn)


def _sqdist(src, dst):
    return (jnp.sum(src ** 2, -1)[:, :, None] + jnp.sum(dst ** 2, -1)[:, None, :]
            - 2.0 * jnp.matmul(src, dst.transpose(0, 2, 1)))


def _gather_pts(points, idx):
    B_ = points.shape[0]
    flat = idx.reshape(B_, -1)
    out = jnp.take_along_axis(points, flat[..., None], axis=1)
    return out.reshape(idx.shape + (points.shape[-1],))


def _ball_idx(radius, K, xyz, new_xyz):
    N_ = xyz.shape[1]
    sqrdists = _sqdist(new_xyz, xyz)
    masked = jnp.where(sqrdists > radius ** 2, N_,
                       jnp.arange(N_, dtype=jnp.int32))
    neg, _ = jax.lax.top_k(-masked, K)
    gidx = -neg
    first = gidx[:, :, :1]
    return jnp.where(gidx == N_, jnp.broadcast_to(first, gidx.shape), gidx)


def _sa_msg(xyz_bcn, points_bcn, npoint, radius_list, nsample_list, branches):
    xyz = xyz_bcn.transpose(0, 2, 1)
    points = points_bcn.transpose(0, 2, 1)
    new_xyz_bcn = _fps_newxyz(xyz_bcn, npoint)
    new_xyz = new_xyz_bcn.transpose(0, 2, 1)
    outs = []
    for radius, K, layers in zip(radius_list, nsample_list, branches):
        gidx = _ball_idx(radius, K, xyz, new_xyz)
        grouped_xyz = _gather_pts(xyz, gidx) - new_xyz[:, :, None, :]
        grouped = jnp.concatenate([_gather_pts(points, gidx), grouped_xyz], -1)
        outs.append(_mlp2d_max(grouped.transpose(0, 3, 2, 1), layers))
    return new_xyz_bcn, jnp.concatenate(outs, axis=1)


def _sa_all(xyz_bcn, points_bcn, layers):
    B_ = xyz_bcn.shape[0]
    x = jnp.concatenate([xyz_bcn, points_bcn], axis=1)
    y, a, c = _mlp_flat(x, layers)
    new_xyz_bcn = jnp.zeros((B_, 3, 1), dtype=xyz_bcn.dtype)
    return new_xyz_bcn, _norm_relu_maxlane(y, a, c)


def _fp(xyz1_bcn, xyz2_bcn, points1_bcn, points2_bcn, layers):
    B_, _, N_ = xyz1_bcn.shape
    S_ = xyz2_bcn.shape[2]
    if S_ == 1:
        interp = jnp.broadcast_to(points2_bcn, (B_, points2_bcn.shape[1], N_))
    else:
        dists = _sqdist(xyz1_bcn.transpose(0, 2, 1), xyz2_bcn.transpose(0, 2, 1))
        negd, idx = jax.lax.top_k(-dists, 3)
        d3 = -negd
        recip = 1.0 / (d3 + 1e-8)
        w = recip / jnp.sum(recip, axis=2, keepdims=True)
        gath = _gather_pts(points2_bcn.transpose(0, 2, 1), idx)
        interp = jnp.sum(gath * w[..., None], axis=2).transpose(0, 2, 1)
    x = jnp.concatenate([points1_bcn, interp], axis=1)
    return _mlp1d(x, layers)


def kernel(xyz, occs, params):
    xyzT = xyz.transpose(0, 2, 1)
    occ_oh = jax.nn.one_hot(occs, _NOCC, dtype=jnp.float32).transpose(0, 2, 1)

    l0_xyz = xyzT
    l0_points = xyzT
    l1_xyz, l1_points = _sa_msg(l0_xyz, l0_points, 512, [0.1, 0.2, 0.4],
                                [32, 64, 128], params['sa1'])
    l2_xyz, l2_points = _sa_msg(l1_xyz, l1_points, 128, [0.4, 0.8],
                                [64, 128], params['sa2'])
    l3_xyz, l3_points = _sa_all(l2_xyz, l2_points, params['sa3'])
    l2_points = _fp(l2_xyz, l3_xyz, l2_points, l3_points, params['fp3'])
    l1_points = _fp(l1_xyz, l2_xyz, l1_points, l2_points, params['fp2'])
    l0_in = jnp.concatenate([occ_oh, l0_xyz, l0_points], axis=1)
    l0_points = _fp(l0_xyz, l1_xyz, l0_in, l1_points, params['fp1'])

    h = params['head1']
    y, st = _conv_stats(l0_points, h['w'], h['b'])
    a, c = _bn_coeffs(st, y.shape[0] * y.shape[2], h['g'], h['be'])
    x = _norm_relu(y, a, c)
    h2 = params['head2']
    out, _ = _conv_stats(x, h2['w'], h2['b'])
    return out


# speedup vs baseline: 3.1999x; 3.0430x over previous
import functools

import jax
import jax.numpy as jnp
from jax.experimental import pallas as pl
from jax.experimental.pallas import tpu as pltpu
from jax.experimental.pallas import tpu_sc as plsc

_NOCC = 2


def _sc_gather_rows(table, idx):
    M = idx.shape[0]
    D = table.shape[1]
    info = plsc.get_sparse_core_info()
    nc = info.num_cores
    nw = nc * info.num_subcores
    rows_pw = M // nw
    chunk = rows_pw
    while chunk * (D + 1) * 4 > (400 << 10):
        chunk //= 2
    n_chunks = rows_pw // chunk
    mesh = plsc.VectorSubcoreMesh(core_axis_name="c", subcore_axis_name="s")

    def body(table_hbm, idx_hbm, out_hbm, idx_v, rows_v, sem):
        wid = jax.lax.axis_index("s") * nc + jax.lax.axis_index("c")
        base = wid * rows_pw
        for j in range(n_chunks):
            off = base + j * chunk
            pltpu.sync_copy(idx_hbm.at[pl.ds(off, chunk)], idx_v)
            pltpu.async_copy(table_hbm.at[idx_v], rows_v, sem).wait()
            pltpu.sync_copy(rows_v, out_hbm.at[pl.ds(off, chunk)])

    return pl.kernel(
        body,
        mesh=mesh,
        out_type=jax.ShapeDtypeStruct((M, D), jnp.float32),
        scratch_types=[
            pltpu.VMEM((chunk,), jnp.int32),
            pltpu.VMEM((chunk, D), jnp.float32),
            pltpu.SemaphoreType.DMA,
        ],
    )(table, idx)


def _sc_group(points, xyz, gidx, new_xyz):
    B_, N_, C = points.shape
    S_, K = gidx.shape[1], gidx.shape[2]
    table = jnp.concatenate([points, xyz], axis=-1).reshape(B_ * N_, C + 3)
    pad = (-(C + 3)) % 128
    if pad:
        table = jnp.pad(table, ((0, 0), (0, pad)))
    idxf = (gidx + (jnp.arange(B_, dtype=jnp.int32) * N_)[:, None, None]
            ).reshape(-1)
    g = _sc_gather_rows(table, idxf).reshape(B_, S_, K, C + 3 + pad)
    return jnp.concatenate(
        [g[..., :C], g[..., C:C + 3] - new_xyz[:, :, None, :]], axis=-1)


def _conv_kern(w_ref, b_ref, x_ref, y_ref, st_ref):
    bi = pl.program_id(0)
    mi = pl.program_id(1)

    @pl.when((bi == 0) & (mi == 0))
    def _():
        st_ref[...] = jnp.zeros_like(st_ref)

    x = x_ref[0]
    w = w_ref[...]
    y = jnp.dot(w, x, preferred_element_type=jnp.float32) + b_ref[...]
    y_ref[0] = y
    s = jnp.sum(y, axis=1)
    sq = jnp.sum(y * y, axis=1)
    st_ref[...] += jnp.stack([s, sq])


def _pick_tm(m, c=0, o=0):
    tm = m
    for t in (8192, 4096, 2048, 1024, 512, 256, 128):
        if m % t == 0:
            tm = t
            break
    while tm > 128 and (c + o) * tm * 8 > (24 << 20):
        tm //= 2
    return tm


def _conv_stats(x, w, bias):
    B_, C, M = x.shape
    O = w.shape[0]
    tm = _pick_tm(M, C, O)
    grid = (B_, M // tm)
    y, st = pl.pallas_call(
        _conv_kern,
        grid=grid,
        in_specs=[
            pl.BlockSpec((O, C), lambda b, m: (0, 0)),
            pl.BlockSpec((O, 1), lambda b, m: (0, 0)),
            pl.BlockSpec((1, C, tm), lambda b, m: (b, 0, m)),
        ],
        out_specs=[
            pl.BlockSpec((1, O, tm), lambda b, m: (b, 0, m)),
            pl.BlockSpec((2, O), lambda b, m: (0, 0)),
        ],
        out_shape=[
            jax.ShapeDtypeStruct((B_, O, M), jnp.float32),
            jax.ShapeDtypeStruct((2, O), jnp.float32),
        ],
    )(w, bias.reshape(O, 1), x)
    return y, st


def _bn_coeffs(st, count, g, be):
    mean = st[0] / count
    var = st[1] / count - mean * mean
    rstd = g / jnp.sqrt(var + 1e-5)
    a = rstd
    c = be - mean * rstd
    return a.reshape(-1, 1), c.reshape(-1, 1)


def _nr_kern(a_ref, c_ref, y_ref, o_ref):
    o_ref[0] = jnp.maximum(y_ref[0] * a_ref[...] + c_ref[...], 0.0)


def _norm_relu(y, a, c):
    B_, O, M = y.shape
    tm = _pick_tm(M)
    return pl.pallas_call(
        _nr_kern,
        grid=(B_, M // tm),
        in_specs=[
            pl.BlockSpec((O, 1), lambda b, m: (0, 0)),
            pl.BlockSpec((O, 1), lambda b, m: (0, 0)),
            pl.BlockSpec((1, O, tm), lambda b, m: (b, 0, m)),
        ],
        out_specs=pl.BlockSpec((1, O, tm), lambda b, m: (b, 0, m)),
        out_shape=jax.ShapeDtypeStruct((B_, O, M), jnp.float32),
    )(a, c, y)


def _nrmaxk_kern(a_ref, c_ref, y_ref, o_ref, *, nk):
    ki = pl.program_id(2)
    m = jnp.max(y_ref[0], axis=1)

    @pl.when(ki == 0)
    def _():
        o_ref[0] = m

    @pl.when(ki != 0)
    def _():
        o_ref[0] = jnp.maximum(o_ref[0], m)

    @pl.when(ki == nk - 1)
    def _():
        o_ref[0] = jnp.maximum(o_ref[0] * a_ref[...] + c_ref[...], 0.0)


def _norm_relu_maxk(y, a, c, K, S):
    B_, O, _ = y.shape
    y4 = y.reshape(B_, O, K, S)
    ts = 128 if S % 128 == 0 else S
    tk = K
    while tk > 1 and O * tk * ts * 4 > (4 << 20):
        tk //= 2
    nk = K // tk
    return pl.pallas_call(
        functools.partial(_nrmaxk_kern, nk=nk),
        grid=(B_, S // ts, nk),
        in_specs=[
            pl.BlockSpec((O, 1), lambda b, s, k: (0, 0)),
            pl.BlockSpec((O, 1), lambda b, s, k: (0, 0)),
            pl.BlockSpec((1, O, tk, ts), lambda b, s, k: (b, 0, k, s)),
        ],
        out_specs=pl.BlockSpec((1, O, ts), lambda b, s, k: (b, 0, s)),
        out_shape=jax.ShapeDtypeStruct((B_, O, S), jnp.float32),
    )(a, c, y4)


def _nrmaxlane_kern(a_ref, c_ref, y_ref, o_ref):
    m = jnp.max(y_ref[0], axis=1, keepdims=True)
    o_ref[0] = jnp.maximum(m * a_ref[...] + c_ref[...], 0.0)


def _norm_relu_maxlane(y, a, c):
    B_, O, M = y.shape
    return pl.pallas_call(
        _nrmaxlane_kern,
        grid=(B_,),
        in_specs=[
            pl.BlockSpec((O, 1), lambda b: (0, 0)),
            pl.BlockSpec((O, 1), lambda b: (0, 0)),
            pl.BlockSpec((1, O, M), lambda b: (b, 0, 0)),
        ],
        out_specs=pl.BlockSpec((1, O, 1), lambda b: (b, 0, 0)),
        out_shape=jax.ShapeDtypeStruct((B_, O, 1), jnp.float32),
    )(a, c, y)


def _mlp_flat(x, layers):
    B_, _, M = x.shape
    h = x
    for li, p in enumerate(layers):
        y, st = _conv_stats(h, p['w'], p['b'])
        a, c = _bn_coeffs(st, B_ * M, p['g'], p['be'])
        if li < len(layers) - 1:
            h = _norm_relu(y, a, c)
        else:
            return y, a, c
    raise AssertionError


def _mlp2d_max(x4, layers):
    B_, C, K, S = x4.shape
    y, a, c = _mlp_flat(x4.reshape(B_, C, K * S), layers)
    return _norm_relu_maxk(y, a, c, K, S)


def _mlp1d(x, layers):
    y, a, c = _mlp_flat(x, layers)
    return _norm_relu(y, a, c)


def _fps_kern(x_ref, o_ref, *, npoint):
    xyz = x_ref[...]
    B_, _, N_ = xyz.shape
    iota = jax.lax.broadcasted_iota(jnp.int32, (B_, 1, N_), 2)
    piota = jax.lax.broadcasted_iota(jnp.int32, (1, 1, npoint), 2)

    def body(i, st):
        dist_min, far, acc = st
        onehot = (iota == far[:, :, None]).astype(jnp.float32)
        cen = jnp.sum(xyz * onehot, axis=2, keepdims=True)
        acc = jnp.where(piota == i, cen, acc)
        d = jnp.sum((xyz - cen) ** 2, axis=1, keepdims=True)
        dist_min = jnp.minimum(dist_min, d)
        far = jnp.argmax(dist_min[:, 0, :], axis=1).astype(jnp.int32)[:, None]
        return dist_min, far, acc

    dist0 = jnp.full((B_, 1, N_), 1e10, jnp.float32)
    far0 = jnp.zeros((B_, 1), jnp.int32)
    acc0 = jnp.zeros((B_, 3, npoint), jnp.float32)
    _, _, acc = jax.lax.fori_loop(0, npoint, body, (dist0, far0, acc0))
    o_ref[...] = acc


def _fps_newxyz(xyz_bcn, npoint):
    B_, _, N_ = xyz_bcn.shape
    return pl.pallas_call(
        functools.partial(_fps_kern, npoint=npoint),
        out_shape=jax.ShapeDtypeStruct((B_, 3, npoint), jnp.float32),
    )(xyz_bcn)


def _sqdist(src, dst):
    return (jnp.sum(src ** 2, -1)[:, :, None] + jnp.sum(dst ** 2, -1)[:, None, :]
            - 2.0 * jnp.matmul(src, dst.transpose(0, 2, 1)))


def _gather_pts(points, idx):
    B_ = points.shape[0]
    flat = idx.reshape(B_, -1)
    out = jnp.take_along_axis(points, flat[..., None], axis=1)
    return out.reshape(idx.shape + (points.shape[-1],))


def _ball_idx(radius, K, xyz, new_xyz):
    N_ = xyz.shape[1]
    sqrdists = _sqdist(new_xyz, xyz)
    masked = jnp.where(sqrdists > radius ** 2, N_,
                       jnp.arange(N_, dtype=jnp.int32))
    neg, _ = jax.lax.top_k(-masked, K)
    gidx = -neg
    first = gidx[:, :, :1]
    return jnp.where(gidx == N_, jnp.broadcast_to(first, gidx.shape), gidx)


def _sa_msg(xyz_bcn, points_bcn, npoint, radius_list, nsample_list, branches):
    xyz = xyz_bcn.transpose(0, 2, 1)
    points = points_bcn.transpose(0, 2, 1)
    new_xyz_bcn = _fps_newxyz(xyz_bcn, npoint)
    new_xyz = new_xyz_bcn.transpose(0, 2, 1)
    outs = []
    for radius, K, layers in zip(radius_list, nsample_list, branches):
        gidx = _ball_idx(radius, K, xyz, new_xyz)
        grouped = _sc_group(points, xyz, gidx, new_xyz)
        outs.append(_mlp2d_max(grouped.transpose(0, 3, 2, 1), layers))
    return new_xyz_bcn, jnp.concatenate(outs, axis=1)


def _sa_all(xyz_bcn, points_bcn, layers):
    B_ = xyz_bcn.shape[0]
    x = jnp.concatenate([xyz_bcn, points_bcn], axis=1)
    y, a, c = _mlp_flat(x, layers)
    new_xyz_bcn = jnp.zeros((B_, 3, 1), dtype=xyz_bcn.dtype)
    return new_xyz_bcn, _norm_relu_maxlane(y, a, c)


def _fp(xyz1_bcn, xyz2_bcn, points1_bcn, points2_bcn, layers):
    B_, _, N_ = xyz1_bcn.shape
    S_ = xyz2_bcn.shape[2]
    if S_ == 1:
        interp = jnp.broadcast_to(points2_bcn, (B_, points2_bcn.shape[1], N_))
    else:
        dists = _sqdist(xyz1_bcn.transpose(0, 2, 1), xyz2_bcn.transpose(0, 2, 1))
        negd, idx = jax.lax.top_k(-dists, 3)
        d3 = -negd
        recip = 1.0 / (d3 + 1e-8)
        w = recip / jnp.sum(recip, axis=2, keepdims=True)
        gath = _gather_pts(points2_bcn.transpose(0, 2, 1), idx)
        interp = jnp.sum(gath * w[..., None], axis=2).transpose(0, 2, 1)
    x = jnp.concatenate([points1_bcn, interp], axis=1)
    return _mlp1d(x, layers)


def kernel(xyz, occs, params):
    xyzT = xyz.transpose(0, 2, 1)
    occ_oh = jax.nn.one_hot(occs, _NOCC, dtype=jnp.float32).transpose(0, 2, 1)

    l0_xyz = xyzT
    l0_points = xyzT
    l1_xyz, l1_points = _sa_msg(l0_xyz, l0_points, 512, [0.1, 0.2, 0.4],
                                [32, 64, 128], params['sa1'])
    l2_xyz, l2_points = _sa_msg(l1_xyz, l1_points, 128, [0.4, 0.8],
                                [64, 128], params['sa2'])
    l3_xyz, l3_points = _sa_all(l2_xyz, l2_points, params['sa3'])
    l2_points = _fp(l2_xyz, l3_xyz, l2_points, l3_points, params['fp3'])
    l1_points = _fp(l1_xyz, l2_xyz, l1_points, l2_points, params['fp2'])
    l0_in = jnp.concatenate([occ_oh, l0_xyz, l0_points], axis=1)
    l0_points = _fp(l0_xyz, l1_xyz, l0_in, l1_points, params['fp1'])

    h = params['head1']
    y, st = _conv_stats(l0_points, h['w'], h['b'])
    a, c = _bn_coeffs(st, y.shape[0] * y.shape[2], h['g'], h['be'])
    x = _norm_relu(y, a, c)
    h2 = params['head2']
    out, _ = _conv_stats(x, h2['w'], h2['b'])
    return out


# speedup vs baseline: 4.0259x; 1.2581x over previous
import functools

import jax
import jax.numpy as jnp
from jax.experimental import pallas as pl
from jax.experimental.pallas import tpu as pltpu
from jax.experimental.pallas import tpu_sc as plsc

_NOCC = 2


def _sc_gather_rows(table, idx):
    M = idx.shape[0]
    D = table.shape[1]
    info = plsc.get_sparse_core_info()
    nc = info.num_cores
    nw = nc * info.num_subcores
    rows_pw = M // nw
    chunk = rows_pw
    while chunk * (D + 1) * 4 > (400 << 10):
        chunk //= 2
    n_chunks = rows_pw // chunk
    mesh = plsc.VectorSubcoreMesh(core_axis_name="c", subcore_axis_name="s")

    def body(table_hbm, idx_hbm, out_hbm, idx_v, rows_v, sem):
        wid = jax.lax.axis_index("s") * nc + jax.lax.axis_index("c")
        base = wid * rows_pw
        for j in range(n_chunks):
            off = base + j * chunk
            pltpu.sync_copy(idx_hbm.at[pl.ds(off, chunk)], idx_v)
            pltpu.async_copy(table_hbm.at[idx_v], rows_v, sem).wait()
            pltpu.sync_copy(rows_v, out_hbm.at[pl.ds(off, chunk)])

    return pl.kernel(
        body,
        mesh=mesh,
        out_type=jax.ShapeDtypeStruct((M, D), jnp.float32),
        scratch_types=[
            pltpu.VMEM((chunk,), jnp.int32),
            pltpu.VMEM((chunk, D), jnp.float32),
            pltpu.SemaphoreType.DMA,
        ],
    )(table, idx)


def _sc_group(points, xyz, gidx, new_xyz):
    B_, N_, C = points.shape
    S_, K = gidx.shape[1], gidx.shape[2]
    table = jnp.concatenate([points, xyz], axis=-1).reshape(B_ * N_, C + 3)
    pad = (-(C + 3)) % 128
    if pad:
        table = jnp.pad(table, ((0, 0), (0, pad)))
    idxf = (gidx + (jnp.arange(B_, dtype=jnp.int32) * N_)[:, None, None]
            ).reshape(-1)
    g = _sc_gather_rows(table, idxf).reshape(B_, S_, K, C + 3 + pad)
    return jnp.concatenate(
        [g[..., :C], g[..., C:C + 3] - new_xyz[:, :, None, :]], axis=-1)


def _conv_kern(w_ref, b_ref, x_ref, y_ref, st_ref):
    bi = pl.program_id(0)
    mi = pl.program_id(1)

    @pl.when((bi == 0) & (mi == 0))
    def _():
        st_ref[...] = jnp.zeros_like(st_ref)

    x = x_ref[0]
    w = w_ref[...]
    y = jnp.dot(w, x, preferred_element_type=jnp.float32) + b_ref[...]
    y_ref[0] = y
    s = jnp.sum(y, axis=1)
    sq = jnp.sum(y * y, axis=1)
    st_ref[...] += jnp.stack([s, sq])


def _conv_pn_kern(w_ref, b_ref, a_ref, c_ref, x_ref, y_ref, st_ref):
    bi = pl.program_id(0)
    mi = pl.program_id(1)

    @pl.when((bi == 0) & (mi == 0))
    def _():
        st_ref[...] = jnp.zeros_like(st_ref)

    x = jnp.maximum(x_ref[0] * a_ref[...] + c_ref[...], 0.0)
    w = w_ref[...]
    y = jnp.dot(w, x, preferred_element_type=jnp.float32) + b_ref[...]
    y_ref[0] = y
    s = jnp.sum(y, axis=1)
    sq = jnp.sum(y * y, axis=1)
    st_ref[...] += jnp.stack([s, sq])


def _pick_tm(m, c=0, o=0):
    tm = m
    for t in (8192, 4096, 2048, 1024, 512, 256, 128):
        if m % t == 0:
            tm = t
            break
    while tm > 128 and (c + o) * tm * 8 > (24 << 20):
        tm //= 2
    return tm


def _conv_stats(x, w, bias, prenorm=None):
    B_, C, M = x.shape
    O = w.shape[0]
    tm = _pick_tm(M, C, O)
    grid = (B_, M // tm)
    specs = [
        pl.BlockSpec((O, C), lambda b, m: (0, 0)),
        pl.BlockSpec((O, 1), lambda b, m: (0, 0)),
    ]
    args = [w, bias.reshape(O, 1)]
    kern = _conv_kern
    if prenorm is not None:
        specs += [pl.BlockSpec((C, 1), lambda b, m: (0, 0)),
                  pl.BlockSpec((C, 1), lambda b, m: (0, 0))]
        args += [prenorm[0], prenorm[1]]
        kern = _conv_pn_kern
    specs.append(pl.BlockSpec((1, C, tm), lambda b, m: (b, 0, m)))
    args.append(x)
    y, st = pl.pallas_call(
        kern,
        grid=grid,
        in_specs=specs,
        out_specs=[
            pl.BlockSpec((1, O, tm), lambda b, m: (b, 0, m)),
            pl.BlockSpec((2, O), lambda b, m: (0, 0)),
        ],
        out_shape=[
            jax.ShapeDtypeStruct((B_, O, M), jnp.float32),
            jax.ShapeDtypeStruct((2, O), jnp.float32),
        ],
    )(*args)
    return y, st


def _bn_coeffs(st, count, g, be):
    mean = st[0] / count
    var = st[1] / count - mean * mean
    rstd = g / jnp.sqrt(var + 1e-5)
    a = rstd
    c = be - mean * rstd
    return a.reshape(-1, 1), c.reshape(-1, 1)


def _nr_kern(a_ref, c_ref, y_ref, o_ref):
    o_ref[0] = jnp.maximum(y_ref[0] * a_ref[...] + c_ref[...], 0.0)


def _norm_relu(y, a, c):
    B_, O, M = y.shape
    tm = _pick_tm(M)
    return pl.pallas_call(
        _nr_kern,
        grid=(B_, M // tm),
        in_specs=[
            pl.BlockSpec((O, 1), lambda b, m: (0, 0)),
            pl.BlockSpec((O, 1), lambda b, m: (0, 0)),
            pl.BlockSpec((1, O, tm), lambda b, m: (b, 0, m)),
        ],
        out_specs=pl.BlockSpec((1, O, tm), lambda b, m: (b, 0, m)),
        out_shape=jax.ShapeDtypeStruct((B_, O, M), jnp.float32),
    )(a, c, y)


def _nrmaxk_kern(a_ref, c_ref, y_ref, o_ref, *, nk):
    ki = pl.program_id(2)
    m = jnp.max(y_ref[0], axis=1)

    @pl.when(ki == 0)
    def _():
        o_ref[0] = m

    @pl.when(ki != 0)
    def _():
        o_ref[0] = jnp.maximum(o_ref[0], m)

    @pl.when(ki == nk - 1)
    def _():
        o_ref[0] = jnp.maximum(o_ref[0] * a_ref[...] + c_ref[...], 0.0)


def _norm_relu_maxk(y, a, c, K, S):
    B_, O, _ = y.shape
    y4 = y.reshape(B_, O, K, S)
    ts = 128 if S % 128 == 0 else S
    tk = K
    while tk > 1 and O * tk * ts * 4 > (4 << 20):
        tk //= 2
    nk = K // tk
    return pl.pallas_call(
        functools.partial(_nrmaxk_kern, nk=nk),
        grid=(B_, S // ts, nk),
        in_specs=[
            pl.BlockSpec((O, 1), lambda b, s, k: (0, 0)),
            pl.BlockSpec((O, 1), lambda b, s, k: (0, 0)),
            pl.BlockSpec((1, O, tk, ts), lambda b, s, k: (b, 0, k, s)),
        ],
        out_specs=pl.BlockSpec((1, O, ts), lambda b, s, k: (b, 0, s)),
        out_shape=jax.ShapeDtypeStruct((B_, O, S), jnp.float32),
    )(a, c, y4)


def _nrmaxlane_kern(a_ref, c_ref, y_ref, o_ref):
    m = jnp.max(y_ref[0], axis=1, keepdims=True)
    o_ref[0] = jnp.maximum(m * a_ref[...] + c_ref[...], 0.0)


def _norm_relu_maxlane(y, a, c):
    B_, O, M = y.shape
    return pl.pallas_call(
        _nrmaxlane_kern,
        grid=(B_,),
        in_specs=[
            pl.BlockSpec((O, 1), lambda b: (0, 0)),
            pl.BlockSpec((O, 1), lambda b: (0, 0)),
            pl.BlockSpec((1, O, M), lambda b: (b, 0, 0)),
        ],
        out_specs=pl.BlockSpec((1, O, 1), lambda b: (b, 0, 0)),
        out_shape=jax.ShapeDtypeStruct((B_, O, 1), jnp.float32),
    )(a, c, y)


def _mlp_flat(x, layers):
    B_, _, M = x.shape
    h, pn = x, None
    for p in layers:
        y, st = _conv_stats(h, p['w'], p['b'], prenorm=pn)
        a, c = _bn_coeffs(st, B_ * M, p['g'], p['be'])
        h, pn = y, (a, c)
    return h, a, c


def _mlp2d_max(x4, layers):
    B_, C, K, S = x4.shape
    y, a, c = _mlp_flat(x4.reshape(B_, C, K * S), layers)
    return _norm_relu_maxk(y, a, c, K, S)


def _mlp1d(x, layers):
    y, a, c = _mlp_flat(x, layers)
    return _norm_relu(y, a, c)


def _fps_kern(x_ref, o_ref, *, npoint):
    xyz = x_ref[...]
    B_, _, N_ = xyz.shape
    iota = jax.lax.broadcasted_iota(jnp.int32, (B_, 1, N_), 2)
    piota = jax.lax.broadcasted_iota(jnp.int32, (1, 1, npoint), 2)

    def body(i, st):
        dist_min, far, acc = st
        onehot = (iota == far[:, :, None]).astype(jnp.float32)
        cen = jnp.sum(xyz * onehot, axis=2, keepdims=True)
        acc = jnp.where(piota == i, cen, acc)
        d = jnp.sum((xyz - cen) ** 2, axis=1, keepdims=True)
        dist_min = jnp.minimum(dist_min, d)
        far = jnp.argmax(dist_min[:, 0, :], axis=1).astype(jnp.int32)[:, None]
        return dist_min, far, acc

    dist0 = jnp.full((B_, 1, N_), 1e10, jnp.float32)
    far0 = jnp.zeros((B_, 1), jnp.int32)
    acc0 = jnp.zeros((B_, 3, npoint), jnp.float32)
    _, _, acc = jax.lax.fori_loop(0, npoint, body, (dist0, far0, acc0))
    o_ref[...] = acc


def _fps_newxyz(xyz_bcn, npoint):
    B_, _, N_ = xyz_bcn.shape
    return pl.pallas_call(
        functools.partial(_fps_kern, npoint=npoint),
        out_shape=jax.ShapeDtypeStruct((B_, 3, npoint), jnp.float32),
    )(xyz_bcn)


def _sqdist(src, dst):
    return (jnp.sum(src ** 2, -1)[:, :, None] + jnp.sum(dst ** 2, -1)[:, None, :]
            - 2.0 * jnp.matmul(src, dst.transpose(0, 2, 1)))


def _gather_pts(points, idx):
    B_ = points.shape[0]
    flat = idx.reshape(B_, -1)
    out = jnp.take_along_axis(points, flat[..., None], axis=1)
    return out.reshape(idx.shape + (points.shape[-1],))


def _ball_idx(radius, K, xyz, new_xyz):
    N_ = xyz.shape[1]
    sqrdists = _sqdist(new_xyz, xyz)
    masked = jnp.where(sqrdists > radius ** 2, N_,
                       jnp.arange(N_, dtype=jnp.int32))
    neg, _ = jax.lax.top_k(-masked, K)
    gidx = -neg
    first = gidx[:, :, :1]
    return jnp.where(gidx == N_, jnp.broadcast_to(first, gidx.shape), gidx)


def _sa_msg(xyz_bcn, points_bcn, npoint, radius_list, nsample_list, branches):
    xyz = xyz_bcn.transpose(0, 2, 1)
    points = points_bcn.transpose(0, 2, 1)
    new_xyz_bcn = _fps_newxyz(xyz_bcn, npoint)
    new_xyz = new_xyz_bcn.transpose(0, 2, 1)
    outs = []
    for radius, K, layers in zip(radius_list, nsample_list, branches):
        gidx = _ball_idx(radius, K, xyz, new_xyz)
        grouped = _sc_group(points, xyz, gidx, new_xyz)
        outs.append(_mlp2d_max(grouped.transpose(0, 3, 2, 1), layers))
    return new_xyz_bcn, jnp.concatenate(outs, axis=1)


def _sa_all(xyz_bcn, points_bcn, layers):
    B_ = xyz_bcn.shape[0]
    x = jnp.concatenate([xyz_bcn, points_bcn], axis=1)
    y, a, c = _mlp_flat(x, layers)
    new_xyz_bcn = jnp.zeros((B_, 3, 1), dtype=xyz_bcn.dtype)
    return new_xyz_bcn, _norm_relu_maxlane(y, a, c)


def _fp(xyz1_bcn, xyz2_bcn, points1_bcn, points2_bcn, layers):
    B_, _, N_ = xyz1_bcn.shape
    S_ = xyz2_bcn.shape[2]
    if S_ == 1:
        interp = jnp.broadcast_to(points2_bcn, (B_, points2_bcn.shape[1], N_))
    else:
        dists = _sqdist(xyz1_bcn.transpose(0, 2, 1), xyz2_bcn.transpose(0, 2, 1))
        siota = jnp.arange(S_, dtype=jnp.int32)
        idxs, vals = [], []
        d = dists
        for _ in range(3):
            i = jnp.argmin(d, axis=2).astype(jnp.int32)
            vals.append(jnp.min(d, axis=2))
            idxs.append(i)
            d = jnp.where(siota == i[..., None], jnp.inf, d)
        idx = jnp.stack(idxs, axis=-1)
        d3 = jnp.stack(vals, axis=-1)
        recip = 1.0 / (d3 + 1e-8)
        w = recip / jnp.sum(recip, axis=2, keepdims=True)
        pts2 = points2_bcn.transpose(0, 2, 1)
        C2 = pts2.shape[2]
        padc = (-C2) % 128
        tbl = pts2.reshape(B_ * S_, C2)
        if padc:
            tbl = jnp.pad(tbl, ((0, 0), (0, padc)))
        idxf = (idx + (jnp.arange(B_, dtype=jnp.int32) * S_)[:, None, None]
                ).reshape(-1)
        gath = _sc_gather_rows(tbl, idxf).reshape(B_, N_, 3, C2 + padc)[..., :C2]
        interp = jnp.sum(gath * w[..., None], axis=2).transpose(0, 2, 1)
    x = jnp.concatenate([points1_bcn, interp], axis=1)
    return _mlp1d(x, layers)


def kernel(xyz, occs, params):
    xyzT = xyz.transpose(0, 2, 1)
    occ_oh = jax.nn.one_hot(occs, _NOCC, dtype=jnp.float32).transpose(0, 2, 1)

    l0_xyz = xyzT
    l0_points = xyzT
    l1_xyz, l1_points = _sa_msg(l0_xyz, l0_points, 512, [0.1, 0.2, 0.4],
                                [32, 64, 128], params['sa1'])
    l2_xyz, l2_points = _sa_msg(l1_xyz, l1_points, 128, [0.4, 0.8],
                                [64, 128], params['sa2'])
    l3_xyz, l3_points = _sa_all(l2_xyz, l2_points, params['sa3'])
    l2_points = _fp(l2_xyz, l3_xyz, l2_points, l3_points, params['fp3'])
    l1_points = _fp(l1_xyz, l2_xyz, l1_points, l2_points, params['fp2'])
    l0_in = jnp.concatenate([occ_oh, l0_xyz, l0_points], axis=1)
    l0_points = _fp(l0_xyz, l1_xyz, l0_in, l1_points, params['fp1'])

    h = params['head1']
    y, st = _conv_stats(l0_points, h['w'], h['b'])
    a, c = _bn_coeffs(st, y.shape[0] * y.shape[2], h['g'], h['be'])
    x = _norm_relu(y, a, c)
    h2 = params['head2']
    out, _ = _conv_stats(x, h2['w'], h2['b'])
    return out
